# baseline (speedup 1.0000x reference)
import functools

import jax
import jax.numpy as jnp
from jax import lax
from jax.experimental import pallas as pl
from jax.experimental.pallas import tpu as pltpu
from jax.experimental.pallas import tpu_sc as plsc

N = 10000
E = 320000
D = 128

NC = 2
NS = 16
NW = NC * NS

C = 128

EPW = E // NW
K = 80
EPW_PAD = K * C
PAD = EPW_PAD - EPW

DEG_SHIFT = 11000
DEG_ROWS = 22528
DEG_ZROWS = DEG_ROWS // NS

NH = N // NC
ACC_H = 5120
EPT = E // NS
K2 = 160
EPT_PAD = K2 * C
PAD2 = EPT_PAD - EPT
ZROWS2 = ACC_H // NS
ZC = 64
ZCH2 = ZROWS2 // ZC

_mesh = plsc.VectorSubcoreMesh(core_axis_name="c", subcore_axis_name="s")


@functools.partial(
    pl.kernel,
    out_type=jax.ShapeDtypeStruct((NC, DEG_ROWS), jnp.float32),
    mesh=_mesh,
    scratch_types=[
        pltpu.VMEM((K, C), jnp.int32),
        pltpu.VMEM((K, C), jnp.int32),
        pltpu.VMEM((C,), jnp.float32),
        pltpu.VMEM((DEG_ZROWS,), jnp.float32),
        pltpu.VMEM_SHARED((DEG_ROWS,), jnp.float32),
        pltpu.SemaphoreType.DMA,
    ],
)
def _deg_kernel(src_hbm, dst_hbm, out_hbm, src_v, dst_v, ones_v, buf_v, acc,
                sem):
    cid = lax.axis_index("c")
    sid = lax.axis_index("s")
    wid = cid * NS + sid
    pltpu.sync_copy(src_hbm.at[wid], src_v)
    pltpu.sync_copy(dst_hbm.at[wid], dst_v)

    @pl.loop(0, C // 16)
    def _ofill(r):
        ones_v[pl.ds(r * 16, 16)] = jnp.ones((16,), jnp.float32)

    @pl.loop(0, DEG_ZROWS // 16)
    def _zfill(r):
        buf_v[pl.ds(r * 16, 16)] = jnp.zeros((16,), jnp.float32)

    pltpu.sync_copy(buf_v, acc.at[pl.ds(sid * DEG_ZROWS, DEG_ZROWS)])
    plsc.subcore_barrier()

    @pl.loop(0, K // 8)
    def _body(g):
        for i in range(8):
            j = g * 8 + i
            pltpu.async_copy(ones_v, acc.at[src_v.at[j]], sem, add=True)
            pltpu.async_copy(ones_v, acc.at[dst_v.at[j]], sem, add=True)
        for i in range(8):
            j = g * 8 + i
            pltpu.make_async_copy(ones_v, acc.at[src_v.at[j]], sem).wait()
            pltpu.make_async_copy(ones_v, acc.at[dst_v.at[j]], sem).wait()

    plsc.subcore_barrier()
    pltpu.sync_copy(acc.at[pl.ds(sid * DEG_ZROWS, DEG_ZROWS)], buf_v)
    pltpu.sync_copy(buf_v, out_hbm.at[cid, pl.ds(sid * DEG_ZROWS, DEG_ZROWS)])


@functools.partial(
    pl.kernel,
    out_type=jax.ShapeDtypeStruct((NC, ACC_H, D), jnp.float32),
    mesh=_mesh,
    scratch_types=[
        pltpu.VMEM((K2, C), jnp.int32),
        pltpu.VMEM((K2, C), jnp.int32),
        pltpu.VMEM((C, D), jnp.float32),
        pltpu.VMEM((C, D), jnp.float32),
        pltpu.VMEM((ZC, D), jnp.float32),
        pltpu.VMEM_SHARED((ACC_H, D), jnp.float32),
        pltpu.SemaphoreType.DMA,
        pltpu.SemaphoreType.DMA,
        pltpu.SemaphoreType.DMA,
        pltpu.SemaphoreType.DMA,
    ],
)
def _agg_kernel(h_hbm, src_hbm, dst_hbm, out_hbm,
                src_v, dst_v, rows_a, rows_b, buf_v, acc,
                sem_ga, sem_gb, sem_sa, sem_sb):
    cid = lax.axis_index("c")
    sid = lax.axis_index("s")
    pltpu.sync_copy(src_hbm.at[sid], src_v)
    pltpu.sync_copy(dst_hbm.at[cid, sid], dst_v)

    @pl.loop(0, ZC)
    def _zfill(r):
        @pl.loop(0, D // 16)
        def _zlane(c):
            buf_v[r, pl.ds(c * 16, 16)] = jnp.zeros((16,), jnp.float32)

    @pl.loop(0, ZCH2)
    def _zacc(k):
        pltpu.sync_copy(buf_v, acc.at[pl.ds(sid * ZROWS2 + k * ZC, ZC)])

    plsc.subcore_barrier()

    pltpu.async_copy(h_hbm.at[src_v.at[0]], rows_a, sem_ga)
    pltpu.make_async_copy(h_hbm.at[src_v.at[0]], rows_a, sem_ga).wait()
    pltpu.async_copy(rows_a, acc.at[dst_v.at[0]], sem_sa, add=True)
    pltpu.async_copy(h_hbm.at[src_v.at[1]], rows_b, sem_gb)

    @pl.loop(0, K2 // 2 - 1)
    def _body(jj):
        j_odd = 2 * jj + 1
        j_even = 2 * jj + 2
        pltpu.make_async_copy(rows_a, acc.at[dst_v.at[0]], sem_sa).wait()
        pltpu.async_copy(h_hbm.at[src_v.at[j_even]], rows_a, sem_ga)
        pltpu.make_async_copy(h_hbm.at[src_v.at[j_odd]], rows_b, sem_gb).wait()
        pltpu.async_copy(rows_b, acc.at[dst_v.at[j_odd]], sem_sb, add=True)
        pltpu.make_async_copy(h_hbm.at[src_v.at[j_even]], rows_a, sem_ga).wait()
        pltpu.async_copy(rows_a, acc.at[dst_v.at[j_even]], sem_sa, add=True)
        pltpu.make_async_copy(rows_b, acc.at[dst_v.at[0]], sem_sb).wait()
        pltpu.async_copy(h_hbm.at[src_v.at[j_odd + 2]], rows_b, sem_gb)

    pltpu.make_async_copy(h_hbm.at[src_v.at[0]], rows_b, sem_gb).wait()
    pltpu.async_copy(rows_b, acc.at[dst_v.at[K2 - 1]], sem_sb, add=True)
    pltpu.make_async_copy(rows_a, acc.at[dst_v.at[0]], sem_sa).wait()
    pltpu.make_async_copy(rows_b, acc.at[dst_v.at[0]], sem_sb).wait()

    plsc.subcore_barrier()

    @pl.loop(0, ZCH2)
    def _cout(k):
        pltpu.sync_copy(acc.at[pl.ds(sid * ZROWS2 + k * ZC, ZC)], buf_v)
        pltpu.sync_copy(buf_v, out_hbm.at[cid, pl.ds(sid * ZROWS2 + k * ZC, ZC)])


_R = 1000


def _prep_body(dego_ref, degi_ref, feat_ref, h0s_ref, ns_ref, nd_ref):
    do = dego_ref[0] + dego_ref[1]
    di = degi_ref[0] + degi_ref[1]
    ns = lax.rsqrt(jnp.maximum(do, 1.0))
    nd = lax.rsqrt(jnp.maximum(di, 1.0))
    ns_ref[...] = ns
    nd_ref[...] = nd
    h0s_ref[...] = feat_ref[...] * ns


_prep_call = pl.pallas_call(
    _prep_body,
    grid=(N // _R,),
    in_specs=[
        pl.BlockSpec((NC, _R, 1), lambda i: (0, i, 0)),
        pl.BlockSpec((NC, _R, 1), lambda i: (0, i, 0)),
        pl.BlockSpec((_R, D), lambda i: (i, 0)),
    ],
    out_specs=[
        pl.BlockSpec((_R, D), lambda i: (i, 0)),
        pl.BlockSpec((_R, 1), lambda i: (i, 0)),
        pl.BlockSpec((_R, 1), lambda i: (i, 0)),
    ],
    out_shape=[
        jax.ShapeDtypeStruct((N, D), jnp.float32),
        jax.ShapeDtypeStruct((N, 1), jnp.float32),
        jax.ShapeDtypeStruct((N, 1), jnp.float32),
    ],
)

_agg_spec = pl.BlockSpec((1, _R, D), lambda i: (i // 5, i % 5, 0))


def _l1_body(acc_ref, nd_ref, w_ref, b_ref, feat_ref, ns_ref, out_ref):
    a = acc_ref[0] * nd_ref[...]
    z = jnp.dot(a, w_ref[...], preferred_element_type=jnp.float32) + b_ref[...]
    h = jnp.maximum(z, 0.0) + feat_ref[...]
    out_ref[...] = h * ns_ref[...]


_l1_call = pl.pallas_call(
    _l1_body,
    grid=(N // _R,),
    in_specs=[
        _agg_spec,
        pl.BlockSpec((_R, 1), lambda i: (i, 0)),
        pl.BlockSpec((D, D), lambda i: (0, 0)),
        pl.BlockSpec((1, D), lambda i: (0, 0)),
        pl.BlockSpec((_R, D), lambda i: (i, 0)),
        pl.BlockSpec((_R, 1), lambda i: (i, 0)),
    ],
    out_specs=pl.BlockSpec((_R, D), lambda i: (i, 0)),
    out_shape=jax.ShapeDtypeStruct((N, D), jnp.float32),
)


def _l2_body(acc_ref, nd_ref, w_ref, b_ref, feat_ref, out_ref):
    a = acc_ref[0] * nd_ref[...]
    z = jnp.dot(a, w_ref[...], preferred_element_type=jnp.float32) + b_ref[...]
    out_ref[...] = jnp.maximum(z, 0.0) + 2.0 * feat_ref[...]


_l2_call = pl.pallas_call(
    _l2_body,
    grid=(N // _R,),
    in_specs=[
        _agg_spec,
        pl.BlockSpec((_R, 1), lambda i: (i, 0)),
        pl.BlockSpec((D, D), lambda i: (0, 0)),
        pl.BlockSpec((1, D), lambda i: (0, 0)),
        pl.BlockSpec((_R, D), lambda i: (i, 0)),
    ],
    out_specs=pl.BlockSpec((_R, D), lambda i: (i, 0)),
    out_shape=jax.ShapeDtypeStruct((N, D), jnp.float32),
)


def kernel(features, edge_index, W1, b1, W2, b2):
    src = edge_index[0]
    dst = edge_index[1]

    src_w = src.reshape(NW, EPW)
    dst_w = dst.reshape(NW, EPW)
    lane = jnp.arange(PAD, dtype=jnp.int32)[None, :]
    widv = jnp.arange(NW, dtype=jnp.int32)[:, None]
    pad_deg_src = 10000 + ((widv * 31 + lane) % 512)
    pad_deg_dst = 21000 + ((widv * 31 + lane) % 1000)

    def cat(a, p, k):
        return jnp.concatenate([a, p.astype(jnp.int32)], axis=-1).reshape(
            a.shape[:-1] + (k, C))

    src_deg = cat(src_w, pad_deg_src, K)
    dst_deg = cat(dst_w + DEG_SHIFT, pad_deg_dst, K)

    src_t = src.reshape(NS, EPT)
    dst_t = dst.reshape(NS, EPT)
    lane2 = jnp.arange(PAD2, dtype=jnp.int32)[None, :]
    sidv = jnp.arange(NS, dtype=jnp.int32)[:, None]
    pad_gather = (sidv * 613 + lane2 * 97) % N
    src_agg = cat(src_t, pad_gather, K2)

    trash = NH + ((jnp.arange(EPT, dtype=jnp.int32)[None, :]
                   + 17 * sidv) % (ACC_H - NH))
    base = jnp.array([0, NH], dtype=jnp.int32)[:, None, None]
    drel = dst_t[None] - base
    dcore = jnp.where((drel >= 0) & (drel < NH), drel, trash[None])
    pad_trash = NH + ((sidv * 7 + lane2) % (ACC_H - NH))
    dst_agg = cat(dcore, jnp.broadcast_to(pad_trash[None], (NC, NS, PAD2)), K2)

    b1r = b1.reshape(1, D)
    b2r = b2.reshape(1, D)

    degp = _deg_kernel(src_deg, dst_deg)
    dego = degp[:, :N].reshape(NC, N, 1)
    degi = degp[:, DEG_SHIFT:DEG_SHIFT + N].reshape(NC, N, 1)
    h0s, ns, nd = _prep_call(dego, degi, features)

    a1 = _agg_kernel(h0s, src_agg, dst_agg)
    h1s = _l1_call(a1, nd, W1, b1r, features, ns)
    a2 = _agg_kernel(h1s, src_agg, dst_agg)
    return _l2_call(a2, nd, W2, b2r, features)


# speedup vs baseline: 1.3088x; 1.3088x over previous
import functools

import jax
import jax.numpy as jnp
from jax import lax
from jax.experimental import pallas as pl
from jax.experimental.pallas import tpu as pltpu
from jax.experimental.pallas import tpu_sc as plsc

N = 10000
E = 320000
D = 128

NC = 2
NS = 16
NW = NC * NS

C = 128

EPW = E // NW
K = 80
EPW_PAD = K * C
PAD = EPW_PAD - EPW

DEG_SHIFT = 11000
DEG_ROWS = 22528
DEG_ZROWS = DEG_ROWS // NS

NH = N // NC
ACC_H = 5120
EPT = E // NS
K2 = 160
EPT_PAD = K2 * C
PAD2 = EPT_PAD - EPT
ZROWS2 = ACC_H // NS
ZC = 64
ZCH2 = ZROWS2 // ZC

_mesh = plsc.VectorSubcoreMesh(core_axis_name="c", subcore_axis_name="s")


@functools.partial(
    pl.kernel,
    out_type=jax.ShapeDtypeStruct((NC, DEG_ROWS), jnp.float32),
    mesh=_mesh,
    scratch_types=[
        pltpu.VMEM((K, C), jnp.int32),
        pltpu.VMEM((K, C), jnp.int32),
        pltpu.VMEM((C,), jnp.float32),
        pltpu.VMEM((DEG_ZROWS,), jnp.float32),
        pltpu.VMEM_SHARED((DEG_ROWS,), jnp.float32),
        pltpu.SemaphoreType.DMA,
    ],
)
def _deg_kernel(src_hbm, dst_hbm, out_hbm, src_v, dst_v, ones_v, buf_v, acc,
                sem):
    cid = lax.axis_index("c")
    sid = lax.axis_index("s")
    wid = cid * NS + sid
    pltpu.sync_copy(src_hbm.at[wid], src_v)
    pltpu.sync_copy(dst_hbm.at[wid], dst_v)

    @pl.loop(0, C // 16)
    def _ofill(r):
        ones_v[pl.ds(r * 16, 16)] = jnp.ones((16,), jnp.float32)

    @pl.loop(0, DEG_ZROWS // 16)
    def _zfill(r):
        buf_v[pl.ds(r * 16, 16)] = jnp.zeros((16,), jnp.float32)

    pltpu.sync_copy(buf_v, acc.at[pl.ds(sid * DEG_ZROWS, DEG_ZROWS)])
    plsc.subcore_barrier()

    @pl.loop(0, K // 8)
    def _body(g):
        for i in range(8):
            j = g * 8 + i
            pltpu.async_copy(ones_v, acc.at[src_v.at[j]], sem, add=True)
            pltpu.async_copy(ones_v, acc.at[dst_v.at[j]], sem, add=True)
        for i in range(8):
            j = g * 8 + i
            pltpu.make_async_copy(ones_v, acc.at[src_v.at[j]], sem).wait()
            pltpu.make_async_copy(ones_v, acc.at[dst_v.at[j]], sem).wait()

    plsc.subcore_barrier()
    pltpu.sync_copy(acc.at[pl.ds(sid * DEG_ZROWS, DEG_ZROWS)], buf_v)
    pltpu.sync_copy(buf_v, out_hbm.at[cid, pl.ds(sid * DEG_ZROWS, DEG_ZROWS)])


@functools.partial(
    pl.kernel,
    out_type=jax.ShapeDtypeStruct((NC, ACC_H, D), jnp.float32),
    mesh=_mesh,
    scratch_types=[
        pltpu.VMEM((K2, C), jnp.int32),
        pltpu.VMEM((K2, C), jnp.int32),
        pltpu.VMEM((C, D), jnp.float32),
        pltpu.VMEM((C, D), jnp.float32),
        pltpu.VMEM((ZC, D), jnp.float32),
        pltpu.VMEM_SHARED((ACC_H, D), jnp.float32),
        pltpu.SemaphoreType.DMA,
        pltpu.SemaphoreType.DMA,
    ],
)
def _agg_kernel(h_hbm, src_hbm, dst_hbm, out_hbm,
                src_v, dst_v, rows_a, rows_b, buf_v, acc, sem_a, sem_b):
    cid = lax.axis_index("c")
    sid = lax.axis_index("s")
    pltpu.sync_copy(src_hbm.at[sid], src_v)
    pltpu.sync_copy(dst_hbm.at[cid, sid], dst_v)

    @pl.loop(0, ZC)
    def _zfill(r):
        @pl.loop(0, D // 16)
        def _zlane(c):
            buf_v[r, pl.ds(c * 16, 16)] = jnp.zeros((16,), jnp.float32)

    @pl.loop(0, ZCH2)
    def _zacc(k):
        pltpu.sync_copy(buf_v, acc.at[pl.ds(sid * ZROWS2 + k * ZC, ZC)])

    plsc.subcore_barrier()

    pltpu.async_copy(h_hbm.at[src_v.at[0]], rows_a, sem_a)

    @pl.loop(0, K2 // 2)
    def _body(jj):
        j0 = jj * 2
        j1 = j0 + 1
        j2 = lax.rem(j0 + 2, K2)
        pltpu.async_copy(h_hbm.at[src_v.at[j1]], rows_b, sem_b)
        pltpu.make_async_copy(h_hbm.at[src_v.at[j0]], rows_a, sem_a).wait()
        pltpu.sync_copy(rows_a, acc.at[dst_v.at[j0]], add=True)
        pltpu.async_copy(h_hbm.at[src_v.at[j2]], rows_a, sem_a)
        pltpu.make_async_copy(h_hbm.at[src_v.at[j1]], rows_b, sem_b).wait()
        pltpu.sync_copy(rows_b, acc.at[dst_v.at[j1]], add=True)

    pltpu.make_async_copy(h_hbm.at[src_v.at[0]], rows_a, sem_a).wait()

    plsc.subcore_barrier()

    @pl.loop(0, ZCH2)
    def _cout(k):
        pltpu.sync_copy(acc.at[pl.ds(sid * ZROWS2 + k * ZC, ZC)], buf_v)
        pltpu.sync_copy(buf_v, out_hbm.at[cid, pl.ds(sid * ZROWS2 + k * ZC, ZC)])


_R = 1000


def _prep_body(dego_ref, degi_ref, feat_ref, h0s_ref, ns_ref, nd_ref):
    do = dego_ref[0] + dego_ref[1]
    di = degi_ref[0] + degi_ref[1]
    ns = lax.rsqrt(jnp.maximum(do, 1.0))
    nd = lax.rsqrt(jnp.maximum(di, 1.0))
    ns_ref[...] = ns
    nd_ref[...] = nd
    h0s_ref[...] = feat_ref[...] * ns


_prep_call = pl.pallas_call(
    _prep_body,
    grid=(N // _R,),
    in_specs=[
        pl.BlockSpec((NC, _R, 1), lambda i: (0, i, 0)),
        pl.BlockSpec((NC, _R, 1), lambda i: (0, i, 0)),
        pl.BlockSpec((_R, D), lambda i: (i, 0)),
    ],
    out_specs=[
        pl.BlockSpec((_R, D), lambda i: (i, 0)),
        pl.BlockSpec((_R, 1), lambda i: (i, 0)),
        pl.BlockSpec((_R, 1), lambda i: (i, 0)),
    ],
    out_shape=[
        jax.ShapeDtypeStruct((N, D), jnp.float32),
        jax.ShapeDtypeStruct((N, 1), jnp.float32),
        jax.ShapeDtypeStruct((N, 1), jnp.float32),
    ],
)

_agg_spec = pl.BlockSpec((1, _R, D), lambda i: (i // 5, i % 5, 0))


def _l1_body(acc_ref, nd_ref, w_ref, b_ref, feat_ref, ns_ref, out_ref):
    a = acc_ref[0] * nd_ref[...]
    z = jnp.dot(a, w_ref[...], preferred_element_type=jnp.float32) + b_ref[...]
    h = jnp.maximum(z, 0.0) + feat_ref[...]
    out_ref[...] = h * ns_ref[...]


_l1_call = pl.pallas_call(
    _l1_body,
    grid=(N // _R,),
    in_specs=[
        _agg_spec,
        pl.BlockSpec((_R, 1), lambda i: (i, 0)),
        pl.BlockSpec((D, D), lambda i: (0, 0)),
        pl.BlockSpec((1, D), lambda i: (0, 0)),
        pl.BlockSpec((_R, D), lambda i: (i, 0)),
        pl.BlockSpec((_R, 1), lambda i: (i, 0)),
    ],
    out_specs=pl.BlockSpec((_R, D), lambda i: (i, 0)),
    out_shape=jax.ShapeDtypeStruct((N, D), jnp.float32),
)


def _l2_body(acc_ref, nd_ref, w_ref, b_ref, feat_ref, out_ref):
    a = acc_ref[0] * nd_ref[...]
    z = jnp.dot(a, w_ref[...], preferred_element_type=jnp.float32) + b_ref[...]
    out_ref[...] = jnp.maximum(z, 0.0) + 2.0 * feat_ref[...]


_l2_call = pl.pallas_call(
    _l2_body,
    grid=(N // _R,),
    in_specs=[
        _agg_spec,
        pl.BlockSpec((_R, 1), lambda i: (i, 0)),
        pl.BlockSpec((D, D), lambda i: (0, 0)),
        pl.BlockSpec((1, D), lambda i: (0, 0)),
        pl.BlockSpec((_R, D), lambda i: (i, 0)),
    ],
    out_specs=pl.BlockSpec((_R, D), lambda i: (i, 0)),
    out_shape=jax.ShapeDtypeStruct((N, D), jnp.float32),
)


def kernel(features, edge_index, W1, b1, W2, b2):
    src = edge_index[0]
    dst = edge_index[1]

    src_w = src.reshape(NW, EPW)
    dst_w = dst.reshape(NW, EPW)
    lane = jnp.arange(PAD, dtype=jnp.int32)[None, :]
    widv = jnp.arange(NW, dtype=jnp.int32)[:, None]
    pad_deg_src = 10000 + ((widv * 31 + lane) % 512)
    pad_deg_dst = 21000 + ((widv * 31 + lane) % 1000)

    def cat(a, p, k):
        return jnp.concatenate([a, p.astype(jnp.int32)], axis=-1).reshape(
            a.shape[:-1] + (k, C))

    src_deg = cat(src_w, pad_deg_src, K)
    dst_deg = cat(dst_w + DEG_SHIFT, pad_deg_dst, K)

    src_t = src.reshape(NS, EPT)
    dst_t = dst.reshape(NS, EPT)
    lane2 = jnp.arange(PAD2, dtype=jnp.int32)[None, :]
    sidv = jnp.arange(NS, dtype=jnp.int32)[:, None]
    pad_gather = (sidv * 613 + lane2 * 97) % N
    src_agg = cat(src_t, pad_gather, K2)

    trash = NH + ((jnp.arange(EPT, dtype=jnp.int32)[None, :]
                   + 17 * sidv) % (ACC_H - NH))
    base = jnp.array([0, NH], dtype=jnp.int32)[:, None, None]
    drel = dst_t[None] - base
    dcore = jnp.where((drel >= 0) & (drel < NH), drel, trash[None])
    pad_trash = NH + ((sidv * 7 + lane2) % (ACC_H - NH))
    dst_agg = cat(dcore, jnp.broadcast_to(pad_trash[None], (NC, NS, PAD2)), K2)

    b1r = b1.reshape(1, D)
    b2r = b2.reshape(1, D)

    degp = _deg_kernel(src_deg, dst_deg)
    dego = degp[:, :N].reshape(NC, N, 1)
    degi = degp[:, DEG_SHIFT:DEG_SHIFT + N].reshape(NC, N, 1)
    h0s, ns, nd = _prep_call(dego, degi, features)

    a1 = _agg_kernel(h0s, src_agg, dst_agg)
    h1s = _l1_call(a1, nd, W1, b1r, features, ns)
    a2 = _agg_kernel(h1s, src_agg, dst_agg)
    return _l2_call(a2, nd, W2, b2r, features)
